# baseline (speedup 1.0000x reference)
import functools

import jax
import jax.numpy as jnp
from jax import lax
from jax.experimental import pallas as pl
from jax.experimental.pallas import tpu as pltpu
from jax.experimental.pallas import tpu_sc as plsc

B, S, D, H, DH, DFF = 2, 4096, 1024, 16, 64, 4096
K = 512
MB = 512
NMB = S // MB
NC, NS = 2, 16
NW = NC * NS
RPW = (B * K) // NW

_f32 = jnp.float32
_i32 = jnp.int32


SCH = S // 8


def _route_select_body(x_ref, rwt_ref, rb_ref, tp_ref, fx_ref, w_ref):
    m = pl.program_id(0)
    for b in range(B):
        wc = lax.dot_general(rwt_ref[...], x_ref[b],
                             (((1,), (1,)), ((), ())),
                             preferred_element_type=_f32)
        w_ref[b, m, :] = wc[0] + rb_ref[0]

    @pl.when(m == S // SCH - 1)
    def _selection():
        NR = S // SCH
        rN = lax.broadcasted_iota(_i32, (SCH, SCH), 0)
        cN = lax.broadcasted_iota(_i32, (SCH, SCH), 1)
        U = (rN <= cN).astype(_f32)
        r8 = lax.broadcasted_iota(_i32, (NR, NR), 0)
        c8 = lax.broadcasted_iota(_i32, (NR, NR), 1)
        Lx = (r8 > c8).astype(_f32)

        def cumsum2d(e):
            cs = jnp.dot(e, U, preferred_element_type=_f32)
            rowsum = jnp.sum(e, axis=1, keepdims=True)
            off = jnp.dot(Lx, rowsum, preferred_element_type=_f32)
            return cs + off

        for b in range(B):
            wb = w_ref[b]
            bits = lax.bitcast_convert_type(wb, _i32)
            keys = jnp.where(bits >= 0, bits, bits ^ jnp.int32(0x7FFFFFFF))

            def step(_, c):
                lo, hi = c
                mid = (lo >> 1) + (hi >> 1) + (lo & hi & 1)
                cnt = jnp.sum((keys > mid).astype(_i32))
                pred = cnt < K
                return (jnp.where(pred, lo, mid), jnp.where(pred, mid, hi))

            lo0 = jnp.int32(-2147483647) - 1
            hi0 = jnp.int32(2147483647)
            _, T = lax.fori_loop(0, 32, step, (lo0, hi0))

            gt = keys > T
            eq = keys == T
            g = jnp.sum(gt.astype(_i32))
            r = (K - g).astype(_f32)
            ecs = cumsum2d(eq.astype(_f32))
            sel = gt | (eq & (ecs <= r))
            rk = cumsum2d(sel.astype(_f32))

            rki = rk.astype(_i32)
            jcol = lax.broadcasted_iota(_i32, (K, SCH), 0)
            tpc = jnp.zeros((K, 1), _f32)
            for u in range(NR):
                cmp = (rki[u:u + 1, :] <= jcol).astype(_f32)
                tpc = tpc + jnp.sum(cmp, axis=1, keepdims=True)
            tpci = tpc.astype(_i32)
            tp_ref[b] = tpci
            fx_ref[b] = tpci + b * S


def _route_select(x, router_w, router_b):
    tp, fx = pl.pallas_call(
        _route_select_body,
        grid=(S // SCH,),
        in_specs=[
            pl.BlockSpec((B, SCH, D), lambda m: (0, m, 0)),
            pl.BlockSpec((1, D), lambda m: (0, 0)),
            pl.BlockSpec(memory_space=pltpu.SMEM),
        ],
        out_specs=[
            pl.BlockSpec((B, K, 1), lambda m: (0, 0, 0)),
            pl.BlockSpec((B, K, 1), lambda m: (0, 0, 0)),
        ],
        out_shape=[
            jax.ShapeDtypeStruct((B, K, 1), _i32),
            jax.ShapeDtypeStruct((B, K, 1), _i32),
        ],
        scratch_shapes=[pltpu.VMEM((B, S // SCH, SCH), _f32)],
        compiler_params=pltpu.CompilerParams(
            vmem_limit_bytes=100 * 1024 * 1024),
    )(x, router_w.reshape(1, D), router_b)
    return tp.reshape(B, K), fx.reshape(B * K)


def _sc_gather(x2d, flat_idx):
    mesh = plsc.VectorSubcoreMesh(core_axis_name="c", subcore_axis_name="s")

    @functools.partial(
        pl.kernel,
        mesh=mesh,
        out_type=jax.ShapeDtypeStruct((B * K, D), _f32),
        scratch_types=[
            pltpu.VMEM((RPW,), _i32),
            pltpu.VMEM((RPW, D), _f32),
            pltpu.SemaphoreType.DMA,
        ],
    )
    def gather_k(x_hbm, idx_hbm, out_hbm, idx_v, rows_v, sem):
        wid = lax.axis_index("s") * NC + lax.axis_index("c")
        base = wid * RPW
        pltpu.sync_copy(idx_hbm.at[pl.ds(base, RPW)], idx_v)
        pltpu.async_copy(x_hbm.at[idx_v], rows_v, sem).wait()
        pltpu.sync_copy(rows_v, out_hbm.at[pl.ds(base, RPW)])

    return gather_k(x2d, flat_idx)


HG = 4
CG = HG * DH
NG = H // HG


def _attn_body(t_ref, wq_ref, bq_ref, wk_ref, bk_ref, wv_ref, bv_ref,
               wo_ref, bo_ref, g_ref, be_ref, o_ref, acc_ref):
    g = pl.program_id(1)
    t = t_ref[0]
    scale = _f32(1.0 / float(DH) ** 0.5)
    q = (jnp.dot(t, wq_ref[...], preferred_element_type=_f32)
         + bq_ref[...]) * scale
    kk = jnp.dot(t, wk_ref[...], preferred_element_type=_f32) + bk_ref[...]
    v = jnp.dot(t, wv_ref[...], preferred_element_type=_f32) + bv_ref[...]

    ri = lax.broadcasted_iota(_i32, (K, K), 0)
    ci = lax.broadcasted_iota(_i32, (K, K), 1)
    causal = ri >= ci

    outs = []
    for h in range(HG):
        sl = slice(h * DH, (h + 1) * DH)
        sc = lax.dot_general(q[:, sl], kk[:, sl], (((1,), (1,)), ((), ())),
                             preferred_element_type=_f32)
        p = jnp.where(causal, jnp.exp(sc), _f32(0.0))
        den = jnp.sum(p, axis=1, keepdims=True)
        outs.append(jnp.dot(p, v[:, sl], preferred_element_type=_f32) / den)
    a = jnp.concatenate(outs, axis=1)
    part = jnp.dot(a, wo_ref[...], preferred_element_type=_f32)

    @pl.when(g == 0)
    def _():
        acc_ref[...] = part

    @pl.when(g != 0)
    def _():
        acc_ref[...] += part

    @pl.when(g == NG - 1)
    def _():
        o = acc_ref[...] + bo_ref[...]
        mu = jnp.mean(o, axis=1, keepdims=True)
        dlt = o - mu
        var = jnp.mean(dlt * dlt, axis=1, keepdims=True)
        o_ref[0] = dlt * lax.rsqrt(var + 1e-5) * g_ref[...] + be_ref[...]


def _attention(tokens3, wq, bq, wk, bk, wv, bv, wo, bo, ln_g, ln_b):
    wcol = lambda b, g: (0, g)
    vcol = lambda b, g: (g,)
    vfull = lambda b, g: (0,)
    return pl.pallas_call(
        _attn_body,
        grid=(B, NG),
        in_specs=[
            pl.BlockSpec((1, K, D), lambda b, g: (b, 0, 0)),
            pl.BlockSpec((D, CG), wcol), pl.BlockSpec((CG,), vcol),
            pl.BlockSpec((D, CG), wcol), pl.BlockSpec((CG,), vcol),
            pl.BlockSpec((D, CG), wcol), pl.BlockSpec((CG,), vcol),
            pl.BlockSpec((CG, D), lambda b, g: (g, 0)),
            pl.BlockSpec((D,), vfull),
            pl.BlockSpec((D,), vfull), pl.BlockSpec((D,), vfull),
        ],
        out_specs=pl.BlockSpec((1, K, D), lambda b, g: (b, 0, 0)),
        out_shape=jax.ShapeDtypeStruct((B, K, D), _f32),
        scratch_shapes=[pltpu.VMEM((K, D), _f32)],
        compiler_params=pltpu.CompilerParams(
            vmem_limit_bytes=100 * 1024 * 1024),
    )(tokens3, wq, bq, wk, bk, wv, bv, wo, bo, ln_g, ln_b)


FC = 512


def _mlp_body(xi_ref, w1_ref, b1_ref, w2_ref, b2_ref, o_ref):
    c = pl.program_id(0)
    h = jnp.dot(xi_ref[...], w1_ref[...], preferred_element_type=_f32) \
        + b1_ref[...]
    gel = jax.nn.gelu(h)
    contrib = jnp.dot(gel, w2_ref[...], preferred_element_type=_f32)

    @pl.when(c == 0)
    def _():
        o_ref[...] = contrib + b2_ref[...]

    @pl.when(c != 0)
    def _():
        o_ref[...] += contrib


def _mlp(x_inner2, w1, b1, w2, b2):
    return pl.pallas_call(
        _mlp_body,
        grid=(DFF // FC,),
        in_specs=[
            pl.BlockSpec((B * K, D), lambda c: (0, 0)),
            pl.BlockSpec((D, FC), lambda c: (0, c)),
            pl.BlockSpec((FC,), lambda c: (c,)),
            pl.BlockSpec((FC, D), lambda c: (c, 0)),
            pl.BlockSpec((D,), lambda c: (0,)),
        ],
        out_specs=pl.BlockSpec((B * K, D), lambda c: (0, 0)),
        out_shape=jax.ShapeDtypeStruct((B * K, D), _f32),
        compiler_params=pltpu.CompilerParams(
            vmem_limit_bytes=100 * 1024 * 1024),
    )(x_inner2, w1, b1, w2, b2)


def _merge_body(x_ref, nr_ref, tp_ref, o_ref):
    m = pl.program_id(1)
    tpv = tp_ref[0, 0, :]
    ri = lax.broadcasted_iota(_i32, (MB, K), 0) + m * MB
    P = (tpv[None, :] == ri).astype(_f32)
    o_ref[0] = x_ref[0] + jnp.dot(P, nr_ref[0], preferred_element_type=_f32)


def _merge(x, processed3, token_pos3):
    return pl.pallas_call(
        _merge_body,
        grid=(B, NMB),
        in_specs=[
            pl.BlockSpec((1, MB, D), lambda b, m: (b, m, 0)),
            pl.BlockSpec((1, K, D), lambda b, m: (b, 0, 0)),
            pl.BlockSpec((1, 1, K), lambda b, m: (b, 0, 0)),
        ],
        out_specs=pl.BlockSpec((1, MB, D), lambda b, m: (b, m, 0)),
        out_shape=jax.ShapeDtypeStruct((B, S, D), _f32),
        compiler_params=pltpu.CompilerParams(
            vmem_limit_bytes=100 * 1024 * 1024),
    )(x, processed3, token_pos3)


def kernel(x, router_w, router_b, pred_w, pred_b, wq, bq, wk, bk, wv, bv,
           wo, bo, ln_g, ln_b, w1, b1, w2, b2):
    token_pos, flat_idx = _route_select(x, router_w, router_b)
    tokens2 = _sc_gather(x.reshape(B * S, D), flat_idx)

    x_inner = _attention(tokens2.reshape(B, K, D), wq, bq, wk, bk, wv, bv,
                         wo, bo, ln_g, ln_b)
    processed = _mlp(x_inner.reshape(B * K, D), w1, b1, w2, b2)
    return _merge(x, processed.reshape(B, K, D), token_pos.reshape(B, 1, K))


# speedup vs baseline: 1.0791x; 1.0791x over previous
import functools

import jax
import jax.numpy as jnp
from jax import lax
from jax.experimental import pallas as pl
from jax.experimental.pallas import tpu as pltpu
from jax.experimental.pallas import tpu_sc as plsc

B, S, D, H, DH, DFF = 2, 4096, 1024, 16, 64, 4096
K = 512
MB = 1024
NMB = S // MB
NC, NS = 2, 16
NW = NC * NS
RPW = (B * K) // NW

_f32 = jnp.float32
_i32 = jnp.int32


SCH = S // 8


def _route_select_body(x_ref, rwt_ref, rb_ref, tp_ref, fx_ref, w_ref):
    m = pl.program_id(0)
    for b in range(B):
        wc = lax.dot_general(rwt_ref[...], x_ref[b],
                             (((1,), (1,)), ((), ())),
                             preferred_element_type=_f32)
        w_ref[b, m, :] = wc[0] + rb_ref[0]

    @pl.when(m == S // SCH - 1)
    def _selection():
        NR = S // SCH
        rN = lax.broadcasted_iota(_i32, (SCH, SCH), 0)
        cN = lax.broadcasted_iota(_i32, (SCH, SCH), 1)
        U = (rN <= cN).astype(_f32)
        r8 = lax.broadcasted_iota(_i32, (NR, NR), 0)
        c8 = lax.broadcasted_iota(_i32, (NR, NR), 1)
        Lx = (r8 > c8).astype(_f32)

        def cumsum2d(e):
            cs = jnp.dot(e, U, preferred_element_type=_f32)
            rowsum = jnp.sum(e, axis=1, keepdims=True)
            off = jnp.dot(Lx, rowsum, preferred_element_type=_f32)
            return cs + off

        for b in range(B):
            wb = w_ref[b]
            bits = lax.bitcast_convert_type(wb, _i32)
            keys = jnp.where(bits >= 0, bits, bits ^ jnp.int32(0x7FFFFFFF))

            def step(_, c):
                lo, hi = c
                mid = (lo >> 1) + (hi >> 1) + (lo & hi & 1)
                cnt = jnp.sum((keys > mid).astype(_i32))
                pred = cnt < K
                return (jnp.where(pred, lo, mid), jnp.where(pred, mid, hi))

            lo0 = jnp.int32(-2147483647) - 1
            hi0 = jnp.int32(2147483647)
            _, T = lax.fori_loop(0, 32, step, (lo0, hi0))

            gt = keys > T
            eq = keys == T
            g = jnp.sum(gt.astype(_i32))
            r = (K - g).astype(_f32)
            ecs = cumsum2d(eq.astype(_f32))
            sel = gt | (eq & (ecs <= r))
            rk = cumsum2d(sel.astype(_f32))

            rki = rk.astype(_i32)
            jcol = lax.broadcasted_iota(_i32, (K, SCH), 0)
            tpc = jnp.zeros((K, 1), _f32)
            for u in range(NR):
                cmp = (rki[u:u + 1, :] <= jcol).astype(_f32)
                tpc = tpc + jnp.sum(cmp, axis=1, keepdims=True)
            tpci = tpc.astype(_i32)
            tp_ref[b] = tpci
            fx_ref[b] = tpci + b * S


def _route_select(x, router_w, router_b):
    tp, fx = pl.pallas_call(
        _route_select_body,
        grid=(S // SCH,),
        in_specs=[
            pl.BlockSpec((B, SCH, D), lambda m: (0, m, 0)),
            pl.BlockSpec((1, D), lambda m: (0, 0)),
            pl.BlockSpec(memory_space=pltpu.SMEM),
        ],
        out_specs=[
            pl.BlockSpec((B, K, 1), lambda m: (0, 0, 0)),
            pl.BlockSpec((B, K, 1), lambda m: (0, 0, 0)),
        ],
        out_shape=[
            jax.ShapeDtypeStruct((B, K, 1), _i32),
            jax.ShapeDtypeStruct((B, K, 1), _i32),
        ],
        scratch_shapes=[pltpu.VMEM((B, S // SCH, SCH), _f32)],
        compiler_params=pltpu.CompilerParams(
            vmem_limit_bytes=100 * 1024 * 1024),
    )(x, router_w.reshape(1, D), router_b)
    return tp.reshape(B, K), fx.reshape(B * K)


def _sc_gather(x2d, flat_idx):
    mesh = plsc.VectorSubcoreMesh(core_axis_name="c", subcore_axis_name="s")

    @functools.partial(
        pl.kernel,
        mesh=mesh,
        out_type=jax.ShapeDtypeStruct((B * K, D), _f32),
        scratch_types=[
            pltpu.VMEM((RPW,), _i32),
            pltpu.VMEM((RPW, D), _f32),
            pltpu.SemaphoreType.DMA,
        ],
    )
    def gather_k(x_hbm, idx_hbm, out_hbm, idx_v, rows_v, sem):
        wid = lax.axis_index("s") * NC + lax.axis_index("c")
        base = wid * RPW
        pltpu.sync_copy(idx_hbm.at[pl.ds(base, RPW)], idx_v)
        pltpu.async_copy(x_hbm.at[idx_v], rows_v, sem).wait()
        pltpu.sync_copy(rows_v, out_hbm.at[pl.ds(base, RPW)])

    return gather_k(x2d, flat_idx)


def _attn_body(t_ref, wq_ref, bq_ref, wk_ref, bk_ref, wv_ref, bv_ref,
               wo_ref, bo_ref, g_ref, be_ref, o_ref):
    t = t_ref[0]
    scale = _f32(1.0 / float(DH) ** 0.5)
    q = (jnp.dot(t, wq_ref[...], preferred_element_type=_f32)
         + bq_ref[...]) * scale
    kk = jnp.dot(t, wk_ref[...], preferred_element_type=_f32) + bk_ref[...]
    v = jnp.dot(t, wv_ref[...], preferred_element_type=_f32) + bv_ref[...]

    ri = lax.broadcasted_iota(_i32, (K, K), 0)
    ci = lax.broadcasted_iota(_i32, (K, K), 1)
    causal = ri >= ci

    outs = []
    for h in range(H):
        sl = slice(h * DH, (h + 1) * DH)
        sc = lax.dot_general(q[:, sl], kk[:, sl], (((1,), (1,)), ((), ())),
                             preferred_element_type=_f32)
        p = jnp.where(causal, jnp.exp(sc), _f32(0.0))
        den = jnp.sum(p, axis=1, keepdims=True)
        outs.append(jnp.dot(p, v[:, sl], preferred_element_type=_f32) / den)
    a = jnp.concatenate(outs, axis=1)
    o = jnp.dot(a, wo_ref[...], preferred_element_type=_f32) + bo_ref[...]
    mu = jnp.mean(o, axis=1, keepdims=True)
    dlt = o - mu
    var = jnp.mean(dlt * dlt, axis=1, keepdims=True)
    o_ref[0] = dlt * lax.rsqrt(var + 1e-5) * g_ref[...] + be_ref[...]


def _attention(tokens3, wq, bq, wk, bk, wv, bv, wo, bo, ln_g, ln_b):
    wfull = lambda b: (0, 0)
    vfull = lambda b: (0,)
    return pl.pallas_call(
        _attn_body,
        grid=(B,),
        in_specs=[
            pl.BlockSpec((1, K, D), lambda b: (b, 0, 0)),
            pl.BlockSpec((D, D), wfull), pl.BlockSpec((D,), vfull),
            pl.BlockSpec((D, D), wfull), pl.BlockSpec((D,), vfull),
            pl.BlockSpec((D, D), wfull), pl.BlockSpec((D,), vfull),
            pl.BlockSpec((D, D), wfull), pl.BlockSpec((D,), vfull),
            pl.BlockSpec((D,), vfull), pl.BlockSpec((D,), vfull),
        ],
        out_specs=pl.BlockSpec((1, K, D), lambda b: (b, 0, 0)),
        out_shape=jax.ShapeDtypeStruct((B, K, D), _f32),
        compiler_params=pltpu.CompilerParams(
            vmem_limit_bytes=100 * 1024 * 1024),
    )(tokens3, wq, bq, wk, bk, wv, bv, wo, bo, ln_g, ln_b)


FC = 512


def _mlp_body(xi_ref, w1_ref, b1_ref, w2_ref, b2_ref, o_ref):
    c = pl.program_id(0)
    h = jnp.dot(xi_ref[...], w1_ref[...], preferred_element_type=_f32) \
        + b1_ref[...]
    gel = jax.nn.gelu(h)
    contrib = jnp.dot(gel, w2_ref[...], preferred_element_type=_f32)

    @pl.when(c == 0)
    def _():
        o_ref[...] = contrib + b2_ref[...]

    @pl.when(c != 0)
    def _():
        o_ref[...] += contrib


def _mlp(x_inner2, w1, b1, w2, b2):
    return pl.pallas_call(
        _mlp_body,
        grid=(DFF // FC,),
        in_specs=[
            pl.BlockSpec((B * K, D), lambda c: (0, 0)),
            pl.BlockSpec((D, FC), lambda c: (0, c)),
            pl.BlockSpec((FC,), lambda c: (c,)),
            pl.BlockSpec((FC, D), lambda c: (c, 0)),
            pl.BlockSpec((D,), lambda c: (0,)),
        ],
        out_specs=pl.BlockSpec((B * K, D), lambda c: (0, 0)),
        out_shape=jax.ShapeDtypeStruct((B * K, D), _f32),
        compiler_params=pltpu.CompilerParams(
            vmem_limit_bytes=100 * 1024 * 1024),
    )(x_inner2, w1, b1, w2, b2)


def _merge_body(x_ref, nr_ref, tp_ref, o_ref):
    m = pl.program_id(1)
    tpv = tp_ref[0, 0, :]
    ri = lax.broadcasted_iota(_i32, (MB, K), 0) + m * MB
    P = (tpv[None, :] == ri).astype(_f32)
    o_ref[0] = x_ref[0] + jnp.dot(P, nr_ref[0], preferred_element_type=_f32)


def _merge(x, processed3, token_pos3):
    return pl.pallas_call(
        _merge_body,
        grid=(B, NMB),
        in_specs=[
            pl.BlockSpec((1, MB, D), lambda b, m: (b, m, 0)),
            pl.BlockSpec((1, K, D), lambda b, m: (b, 0, 0)),
            pl.BlockSpec((1, 1, K), lambda b, m: (b, 0, 0)),
        ],
        out_specs=pl.BlockSpec((1, MB, D), lambda b, m: (b, m, 0)),
        out_shape=jax.ShapeDtypeStruct((B, S, D), _f32),
        compiler_params=pltpu.CompilerParams(
            vmem_limit_bytes=100 * 1024 * 1024),
    )(x, processed3, token_pos3)


def kernel(x, router_w, router_b, pred_w, pred_b, wq, bq, wk, bk, wv, bv,
           wo, bo, ln_g, ln_b, w1, b1, w2, b2):
    token_pos, flat_idx = _route_select(x, router_w, router_b)
    tokens2 = _sc_gather(x.reshape(B * S, D), flat_idx)

    x_inner = _attention(tokens2.reshape(B, K, D), wq, bq, wk, bk, wv, bv,
                         wo, bo, ln_g, ln_b)
    processed = _mlp(x_inner.reshape(B * K, D), w1, b1, w2, b2)
    return _merge(x, processed.reshape(B, K, D), token_pos.reshape(B, 1, K))


# speedup vs baseline: 1.0916x; 1.0116x over previous
import functools

import jax
import jax.numpy as jnp
from jax import lax
from jax.experimental import pallas as pl
from jax.experimental.pallas import tpu as pltpu
from jax.experimental.pallas import tpu_sc as plsc

B, S, D, H, DH, DFF = 2, 4096, 1024, 16, 64, 4096
K = 512
MB = 1024
NMB = S // MB
NC, NS = 2, 16
NW = NC * NS
RPW = (B * K) // NW

_f32 = jnp.float32
_i32 = jnp.int32


SCH = S // 8


def _route_select_body(x_ref, rwt_ref, rb_ref, tp_ref, fx_ref, w_ref):
    m = pl.program_id(0)
    for b in range(B):
        wc = lax.dot_general(rwt_ref[...], x_ref[b],
                             (((1,), (1,)), ((), ())),
                             preferred_element_type=_f32)
        w_ref[b, m, :] = wc[0] + rb_ref[0]

    @pl.when(m == S // SCH - 1)
    def _selection():
        NR = S // SCH
        rN = lax.broadcasted_iota(_i32, (SCH, SCH), 0)
        cN = lax.broadcasted_iota(_i32, (SCH, SCH), 1)
        U = (rN <= cN).astype(_f32)
        r8 = lax.broadcasted_iota(_i32, (NR, NR), 0)
        c8 = lax.broadcasted_iota(_i32, (NR, NR), 1)
        Lx = (r8 > c8).astype(_f32)

        def cumsum2d(e):
            cs = jnp.dot(e, U, preferred_element_type=_f32)
            rowsum = jnp.sum(e, axis=1, keepdims=True)
            off = jnp.dot(Lx, rowsum, preferred_element_type=_f32)
            return cs + off

        for b in range(B):
            wb = w_ref[b]
            bits = lax.bitcast_convert_type(wb, _i32)
            keys = jnp.where(bits >= 0, bits, bits ^ jnp.int32(0x7FFFFFFF))

            def step(_, c):
                lo, hi = c
                mid = (lo >> 1) + (hi >> 1) + (lo & hi & 1)
                cnt = jnp.sum((keys > mid).astype(_i32))
                pred = cnt < K
                return (jnp.where(pred, lo, mid), jnp.where(pred, mid, hi))

            lo0 = jnp.int32(-2147483647) - 1
            hi0 = jnp.int32(2147483647)
            _, T = lax.fori_loop(0, 32, step, (lo0, hi0))

            gt = keys > T
            eq = keys == T
            g = jnp.sum(gt.astype(_i32))
            r = (K - g).astype(_f32)
            ecs = cumsum2d(eq.astype(_f32))
            sel = gt | (eq & (ecs <= r))
            rk = cumsum2d(sel.astype(_f32))

            rki = rk.astype(_i32)
            jcol = lax.broadcasted_iota(_i32, (K, SCH), 0)
            tpc = jnp.zeros((K, 1), _f32)
            for u in range(NR):
                cmp = (rki[u:u + 1, :] <= jcol).astype(_f32)
                tpc = tpc + jnp.sum(cmp, axis=1, keepdims=True)
            tpci = tpc.astype(_i32)
            tp_ref[b] = tpci
            fx_ref[b] = tpci + b * S


def _route_select(x, router_w, router_b):
    tp, fx = pl.pallas_call(
        _route_select_body,
        grid=(S // SCH,),
        in_specs=[
            pl.BlockSpec((B, SCH, D), lambda m: (0, m, 0)),
            pl.BlockSpec((1, D), lambda m: (0, 0)),
            pl.BlockSpec(memory_space=pltpu.SMEM),
        ],
        out_specs=[
            pl.BlockSpec((B, K, 1), lambda m: (0, 0, 0)),
            pl.BlockSpec((B, K, 1), lambda m: (0, 0, 0)),
        ],
        out_shape=[
            jax.ShapeDtypeStruct((B, K, 1), _i32),
            jax.ShapeDtypeStruct((B, K, 1), _i32),
        ],
        scratch_shapes=[pltpu.VMEM((B, S // SCH, SCH), _f32)],
        compiler_params=pltpu.CompilerParams(
            vmem_limit_bytes=100 * 1024 * 1024),
    )(x, router_w.reshape(1, D), router_b)
    return tp.reshape(B, K), fx.reshape(B * K)


def _sc_gather(x2d, flat_idx):
    mesh = plsc.VectorSubcoreMesh(core_axis_name="c", subcore_axis_name="s")

    @functools.partial(
        pl.kernel,
        mesh=mesh,
        out_type=jax.ShapeDtypeStruct((B * K, D), _f32),
        scratch_types=[
            pltpu.VMEM((RPW,), _i32),
            pltpu.VMEM((RPW, D), _f32),
            pltpu.SemaphoreType.DMA,
        ],
    )
    def gather_k(x_hbm, idx_hbm, out_hbm, idx_v, rows_v, sem):
        wid = lax.axis_index("s") * NC + lax.axis_index("c")
        base = wid * RPW
        pltpu.sync_copy(idx_hbm.at[pl.ds(base, RPW)], idx_v)
        pltpu.async_copy(x_hbm.at[idx_v], rows_v, sem).wait()
        pltpu.sync_copy(rows_v, out_hbm.at[pl.ds(base, RPW)])

    return gather_k(x2d, flat_idx)


def _attn_body(t_ref, wq_ref, bq_ref, wk_ref, bk_ref, wv_ref, bv_ref,
               wo_ref, bo_ref, g_ref, be_ref, o_ref):
    t = t_ref[0]
    scale = _f32(1.0 / float(DH) ** 0.5)
    q = (jnp.dot(t, wq_ref[...], preferred_element_type=_f32)
         + bq_ref[...]) * scale
    kk = jnp.dot(t, wk_ref[...], preferred_element_type=_f32) + bk_ref[...]
    v = jnp.dot(t, wv_ref[...], preferred_element_type=_f32) + bv_ref[...]

    ri = lax.broadcasted_iota(_i32, (K, K), 0)
    ci = lax.broadcasted_iota(_i32, (K, K), 1)
    causal = ri >= ci

    outs = []
    for h in range(H):
        sl = slice(h * DH, (h + 1) * DH)
        sc = lax.dot_general(q[:, sl], kk[:, sl], (((1,), (1,)), ((), ())),
                             preferred_element_type=_f32)
        p = jnp.where(causal, jnp.exp(sc), _f32(0.0))
        den = jnp.sum(p, axis=1, keepdims=True)
        outs.append(jnp.dot(p, v[:, sl], preferred_element_type=_f32) / den)
    a = jnp.concatenate(outs, axis=1)
    o = jnp.dot(a, wo_ref[...], preferred_element_type=_f32) + bo_ref[...]
    mu = jnp.mean(o, axis=1, keepdims=True)
    dlt = o - mu
    var = jnp.mean(dlt * dlt, axis=1, keepdims=True)
    o_ref[0] = dlt * lax.rsqrt(var + 1e-5) * g_ref[...] + be_ref[...]


def _attention(tokens3, wq, bq, wk, bk, wv, bv, wo, bo, ln_g, ln_b):
    wfull = lambda b: (0, 0)
    vfull = lambda b: (0,)
    return pl.pallas_call(
        _attn_body,
        grid=(B,),
        in_specs=[
            pl.BlockSpec((1, K, D), lambda b: (b, 0, 0)),
            pl.BlockSpec((D, D), wfull), pl.BlockSpec((D,), vfull),
            pl.BlockSpec((D, D), wfull), pl.BlockSpec((D,), vfull),
            pl.BlockSpec((D, D), wfull), pl.BlockSpec((D,), vfull),
            pl.BlockSpec((D, D), wfull), pl.BlockSpec((D,), vfull),
            pl.BlockSpec((D,), vfull), pl.BlockSpec((D,), vfull),
        ],
        out_specs=pl.BlockSpec((1, K, D), lambda b: (b, 0, 0)),
        out_shape=jax.ShapeDtypeStruct((B, K, D), _f32),
        compiler_params=pltpu.CompilerParams(
            vmem_limit_bytes=100 * 1024 * 1024),
    )(tokens3, wq, bq, wk, bk, wv, bv, wo, bo, ln_g, ln_b)


FC = 1024


def _mlp_body(xi_ref, w1_ref, b1_ref, w2_ref, b2_ref, o_ref):
    c = pl.program_id(0)
    h = jnp.dot(xi_ref[...], w1_ref[...], preferred_element_type=_f32) \
        + b1_ref[...]
    gel = jax.nn.gelu(h)
    contrib = jnp.dot(gel, w2_ref[...], preferred_element_type=_f32)

    @pl.when(c == 0)
    def _():
        o_ref[...] = contrib + b2_ref[...]

    @pl.when(c != 0)
    def _():
        o_ref[...] += contrib


def _mlp(x_inner2, w1, b1, w2, b2):
    return pl.pallas_call(
        _mlp_body,
        grid=(DFF // FC,),
        in_specs=[
            pl.BlockSpec((B * K, D), lambda c: (0, 0)),
            pl.BlockSpec((D, FC), lambda c: (0, c)),
            pl.BlockSpec((FC,), lambda c: (c,)),
            pl.BlockSpec((FC, D), lambda c: (c, 0)),
            pl.BlockSpec((D,), lambda c: (0,)),
        ],
        out_specs=pl.BlockSpec((B * K, D), lambda c: (0, 0)),
        out_shape=jax.ShapeDtypeStruct((B * K, D), _f32),
        compiler_params=pltpu.CompilerParams(
            vmem_limit_bytes=100 * 1024 * 1024),
    )(x_inner2, w1, b1, w2, b2)


def _merge_body(x_ref, nr_ref, tp_ref, o_ref):
    m = pl.program_id(1)
    tpv = tp_ref[0, 0, :]
    ri = lax.broadcasted_iota(_i32, (MB, K), 0) + m * MB
    P = (tpv[None, :] == ri).astype(_f32)
    o_ref[0] = x_ref[0] + jnp.dot(P, nr_ref[0], preferred_element_type=_f32)


def _merge(x, processed3, token_pos3):
    return pl.pallas_call(
        _merge_body,
        grid=(B, NMB),
        in_specs=[
            pl.BlockSpec((1, MB, D), lambda b, m: (b, m, 0)),
            pl.BlockSpec((1, K, D), lambda b, m: (b, 0, 0)),
            pl.BlockSpec((1, 1, K), lambda b, m: (b, 0, 0)),
        ],
        out_specs=pl.BlockSpec((1, MB, D), lambda b, m: (b, m, 0)),
        out_shape=jax.ShapeDtypeStruct((B, S, D), _f32),
        compiler_params=pltpu.CompilerParams(
            vmem_limit_bytes=100 * 1024 * 1024),
    )(x, processed3, token_pos3)


def kernel(x, router_w, router_b, pred_w, pred_b, wq, bq, wk, bk, wv, bv,
           wo, bo, ln_g, ln_b, w1, b1, w2, b2):
    token_pos, flat_idx = _route_select(x, router_w, router_b)
    tokens2 = _sc_gather(x.reshape(B * S, D), flat_idx)

    x_inner = _attention(tokens2.reshape(B, K, D), wq, bq, wk, bk, wv, bv,
                         wo, bo, ln_g, ln_b)
    processed = _mlp(x_inner.reshape(B * K, D), w1, b1, w2, b2)
    return _merge(x, processed.reshape(B, K, D), token_pos.reshape(B, 1, K))
